# Initial kernel scaffold; baseline (speedup 1.0000x reference)
import functools

import jax
import jax.numpy as jnp
from jax import lax
from jax.experimental import pallas as pl
from jax.experimental.pallas import tpu as pltpu
from jax.experimental.pallas import tpu_sc as plsc

PASS_TIME = 1.0
N_INTERVAL = 100
OUT_DIM = 64
NBINS = N_INTERVAL + 1

NW = 32
GSTEP = 128
NG = 5
MC = GSTEP * NG


def _sc_time_encode(ts_a, ts_b, table, rows):
    rpw = rows // NW
    nmc = rpw // MC
    mesh = plsc.VectorSubcoreMesh(core_axis_name="c", subcore_axis_name="s")

    @functools.partial(
        pl.kernel,
        mesh=mesh,
        out_type=jax.ShapeDtypeStruct((rows, OUT_DIM), jnp.float32),
        scratch_types=[
            pltpu.VMEM((MC,), jnp.float32),
            pltpu.VMEM((MC,), jnp.float32),
            pltpu.VMEM((MC,), jnp.int32),
            pltpu.VMEM((MC, OUT_DIM), jnp.float32),
            pltpu.SemaphoreType.DMA,
        ],
    )
    def k(a_hbm, b_hbm, tab_hbm, out_hbm, a_v, b_v, idx_v, rows_v, sem):
        wid = lax.axis_index("s") * 2 + lax.axis_index("c")
        base = wid * rpw

        def chunk(m, carry):
            off = base + m * MC
            pltpu.sync_copy(a_hbm.at[pl.ds(off, MC)], a_v)
            pltpu.sync_copy(b_hbm.at[pl.ds(off, MC)], b_v)

            def vec(j, c):
                s = pl.ds(j * 16, 16)
                dt = b_v[s] - a_v[s]
                q = (dt * (N_INTERVAL / PASS_TIME)).astype(jnp.int32)
                idx_v[s] = jnp.minimum(jnp.maximum(q, 0), N_INTERVAL)
                return c

            lax.fori_loop(0, MC // 16, vec, 0)

            def gath(g, c):
                cp = pltpu.async_copy(
                    tab_hbm.at[idx_v.at[pl.ds(g * GSTEP, GSTEP)]],
                    rows_v.at[pl.ds(g * GSTEP, GSTEP), :],
                    sem,
                )
                cp.wait()
                return c

            lax.fori_loop(0, NG, gath, 0)
            pltpu.sync_copy(rows_v, out_hbm.at[pl.ds(off, MC), :])
            return carry

        lax.fori_loop(0, nmc, chunk, 0)

    return k(ts_a, ts_b, table)


def kernel(inputs, timestamp, train, W, b):
    batch, L = inputs.shape
    rows = batch * L
    table = W.T + b[None, :]
    ts_prev = timestamp[:, :-1]
    ts_a = ts_prev.reshape(rows)
    ts_b = timestamp[:, 1:].reshape(rows)
    out = _sc_time_encode(ts_a, ts_b, table, rows)
    return out.reshape(batch, L, OUT_DIM), ts_prev


# speedup vs baseline: 3.2667x; 3.2667x over previous
import functools

import jax
import jax.numpy as jnp
from jax import lax
from jax.experimental import pallas as pl
from jax.experimental.pallas import tpu as pltpu
from jax.experimental.pallas import tpu_sc as plsc

PASS_TIME = 1.0
N_INTERVAL = 100
OUT_DIM = 64
NBINS = N_INTERVAL + 1

NW = 32
GSTEP = 128
NG = 4
NPAIR = GSTEP * NG


def _sc_time_encode(ae, ao, be, bo, table2, rows):
    pairs = rows // 2
    ppw = pairs // NW
    nmc = ppw // NPAIR
    mesh = plsc.VectorSubcoreMesh(core_axis_name="c", subcore_axis_name="s")

    @functools.partial(
        pl.kernel,
        mesh=mesh,
        out_type=jax.ShapeDtypeStruct((pairs, 2 * OUT_DIM), jnp.float32),
        scratch_types=[
            pltpu.VMEM((NPAIR,), jnp.float32),
            pltpu.VMEM((NPAIR,), jnp.float32),
            pltpu.VMEM((NPAIR,), jnp.float32),
            pltpu.VMEM((NPAIR,), jnp.float32),
            pltpu.VMEM((NPAIR,), jnp.int32),
            pltpu.VMEM((NPAIR, 2 * OUT_DIM), jnp.float32),
            pltpu.SemaphoreType.DMA,
        ],
    )
    def k(ae_h, ao_h, be_h, bo_h, tab_hbm, out_hbm,
          ae_v, ao_v, be_v, bo_v, idx2_v, rows_v, sem):
        wid = lax.axis_index("s") * 2 + lax.axis_index("c")
        base = wid * ppw

        def chunk(m, carry):
            off = pl.multiple_of(base + m * NPAIR, NPAIR)
            pltpu.sync_copy(ae_h.at[pl.ds(off, NPAIR)], ae_v)
            pltpu.sync_copy(ao_h.at[pl.ds(off, NPAIR)], ao_v)
            pltpu.sync_copy(be_h.at[pl.ds(off, NPAIR)], be_v)
            pltpu.sync_copy(bo_h.at[pl.ds(off, NPAIR)], bo_v)

            def vec(j, c):
                s = pl.ds(j * 16, 16)
                qe = ((be_v[s] - ae_v[s]) * (N_INTERVAL / PASS_TIME)).astype(jnp.int32)
                qe = jnp.minimum(jnp.maximum(qe, 0), N_INTERVAL)
                qo = ((bo_v[s] - ao_v[s]) * (N_INTERVAL / PASS_TIME)).astype(jnp.int32)
                qo = jnp.minimum(jnp.maximum(qo, 0), N_INTERVAL)
                idx2_v[s] = qe * NBINS + qo
                return c

            lax.fori_loop(0, NPAIR // 16, vec, 0)

            def gath(g, c):
                cp = pltpu.async_copy(
                    tab_hbm.at[idx2_v.at[pl.ds(g * GSTEP, GSTEP)]],
                    rows_v.at[pl.ds(g * GSTEP, GSTEP), :],
                    sem,
                )
                cp.wait()
                return c

            lax.fori_loop(0, NG, gath, 0)
            pltpu.sync_copy(rows_v, out_hbm.at[pl.ds(off, NPAIR), :])
            return carry

        lax.fori_loop(0, nmc, chunk, 0)

    return k(ae, ao, be, bo, table2)


def kernel(inputs, timestamp, train, W, b):
    batch, L = inputs.shape
    rows = batch * L
    table = W.T + b[None, :]
    t2 = jnp.concatenate(
        [
            jnp.broadcast_to(table[:, None, :], (NBINS, NBINS, OUT_DIM)),
            jnp.broadcast_to(table[None, :, :], (NBINS, NBINS, OUT_DIM)),
        ],
        axis=-1,
    ).reshape(NBINS * NBINS, 2 * OUT_DIM)
    ts_prev = timestamp[:, :-1]
    ts_a = ts_prev.reshape(rows)
    ts_b = timestamp[:, 1:].reshape(rows)
    ae, ao = ts_a[0::2], ts_a[1::2]
    be, bo = ts_b[0::2], ts_b[1::2]
    out = _sc_time_encode(ae, ao, be, bo, t2, rows)
    return out.reshape(batch, L, OUT_DIM), ts_prev


# speedup vs baseline: 5.3220x; 1.6292x over previous
import functools

import jax
import jax.numpy as jnp
from jax import lax
from jax.experimental import pallas as pl
from jax.experimental.pallas import tpu as pltpu
from jax.experimental.pallas import tpu_sc as plsc

PASS_TIME = 1.0
N_INTERVAL = 100
OUT_DIM = 64
NBINS = N_INTERVAL + 1

NW = 32
CHUNK = 512
NGROUP = CHUNK // 16


def _sc_time_encode(ts_a, ts_b, table_flat, rows):
    rpw = rows // NW
    nchunks = rpw // CHUNK
    npairs = nchunks // 2
    tabn = NBINS * OUT_DIM
    mesh = plsc.VectorSubcoreMesh(core_axis_name="c", subcore_axis_name="s")

    @functools.partial(
        pl.kernel,
        mesh=mesh,
        out_type=jax.ShapeDtypeStruct((rows * OUT_DIM,), jnp.float32),
        scratch_types=[
            pltpu.VMEM((rpw,), jnp.float32),
            pltpu.VMEM((rpw,), jnp.float32),
            pltpu.VMEM((tabn,), jnp.float32),
            pltpu.VMEM((CHUNK * OUT_DIM,), jnp.float32),
            pltpu.VMEM((CHUNK * OUT_DIM,), jnp.float32),
            pltpu.SemaphoreType.DMA,
            pltpu.SemaphoreType.DMA,
        ],
        compiler_params=pltpu.CompilerParams(needs_layout_passes=False),
    )
    def k(a_h, b_h, tab_h, out_h, a_v, b_v, tab_v, bufa, bufb, sema, semb):
        wid = lax.axis_index("s") * 2 + lax.axis_index("c")
        base = pl.multiple_of(wid * rpw, rpw)
        pltpu.sync_copy(a_h.at[pl.ds(base, rpw)], a_v)
        pltpu.sync_copy(b_h.at[pl.ds(base, rpw)], b_v)
        pltpu.sync_copy(tab_h, tab_v)
        row64 = lax.iota(jnp.int32, 16) * OUT_DIM

        def assemble(c_idx, buf):
            roff = pl.multiple_of(c_idx * CHUNK, CHUNK)

            def group(g, carry):
                s = pl.ds(pl.multiple_of(roff + g * 16, 16), 16)
                dt = b_v[s] - a_v[s]
                q = (dt * (N_INTERVAL / PASS_TIME)).astype(jnp.int32)
                idx = jnp.minimum(jnp.maximum(q, 0), N_INTERVAL)
                src = idx * OUT_DIM
                dst = row64 + g * (16 * OUT_DIM)
                for c in range(OUT_DIM):
                    val = plsc.load_gather(tab_v, [src + c])
                    plsc.store_scatter(buf, [dst + c], val)
                return carry

            lax.fori_loop(0, NGROUP, group, 0)

        def start_store(c_idx, buf, sem):
            off = pl.multiple_of((base + c_idx * CHUNK) * OUT_DIM, CHUNK * OUT_DIM)
            return pltpu.async_copy(
                buf, out_h.at[pl.ds(off, CHUNK * OUT_DIM)], sem
            )

        def wait_store(buf, sem):
            pltpu.make_async_copy(
                buf, out_h.at[pl.ds(base * OUT_DIM, CHUNK * OUT_DIM)], sem
            ).wait()

        assemble(0, bufa)
        start_store(0, bufa, sema)
        assemble(1, bufb)
        start_store(1, bufb, semb)

        def pair(p, carry):
            wait_store(bufa, sema)
            assemble(2 * p, bufa)
            start_store(2 * p, bufa, sema)
            wait_store(bufb, semb)
            assemble(2 * p + 1, bufb)
            start_store(2 * p + 1, bufb, semb)
            return carry

        lax.fori_loop(1, npairs, pair, 0)
        wait_store(bufa, sema)
        wait_store(bufb, semb)

    return k(ts_a, ts_b, table_flat)


def kernel(inputs, timestamp, train, W, b):
    batch, L = inputs.shape
    rows = batch * L
    table = (W.T + b[None, :]).reshape(NBINS * OUT_DIM)
    ts_prev = timestamp[:, :-1]
    ts_a = ts_prev.reshape(rows)
    ts_b = timestamp[:, 1:].reshape(rows)
    out = _sc_time_encode(ts_a, ts_b, table, rows)
    return out.reshape(batch, L, OUT_DIM), ts_prev


# speedup vs baseline: 7.1053x; 1.3351x over previous
import functools

import jax
import jax.numpy as jnp
from jax import lax
from jax.experimental import pallas as pl
from jax.experimental.pallas import tpu as pltpu
from jax.experimental.pallas import tpu_sc as plsc

PASS_TIME = 1.0
N_INTERVAL = 100
OUT_DIM = 64
NBINS = N_INTERVAL + 1

NW = 32
CHUNK = 512
NGROUP = CHUNK // 16


def _sc_time_encode(ts_a, ts_b, table_flat, rows):
    rpw = rows // NW
    nchunks = rpw // CHUNK
    npairs = nchunks // 2
    tabn = NBINS * OUT_DIM
    mesh = plsc.VectorSubcoreMesh(core_axis_name="c", subcore_axis_name="s")

    @functools.partial(
        pl.kernel,
        mesh=mesh,
        out_type=jax.ShapeDtypeStruct((rows * OUT_DIM,), jnp.float32),
        scratch_types=[
            pltpu.VMEM((rpw,), jnp.float32),
            pltpu.VMEM((rpw,), jnp.float32),
            pltpu.VMEM((tabn,), jnp.float32),
            pltpu.VMEM((CHUNK * OUT_DIM,), jnp.float32),
            pltpu.VMEM((CHUNK * OUT_DIM,), jnp.float32),
            pltpu.SemaphoreType.DMA,
            pltpu.SemaphoreType.DMA,
        ],
        compiler_params=pltpu.CompilerParams(needs_layout_passes=False),
    )
    def k(a_h, b_h, tab_h, out_h, a_v, b_v, tab_v, bufa, bufb, sema, semb):
        wid = lax.axis_index("s") * 2 + lax.axis_index("c")
        base = pl.multiple_of(wid * rpw, rpw)
        pltpu.sync_copy(a_h.at[pl.ds(base, rpw)], a_v)
        pltpu.sync_copy(b_h.at[pl.ds(base, rpw)], b_v)
        pltpu.sync_copy(tab_h, tab_v)
        row64 = lax.iota(jnp.int32, 16) * OUT_DIM

        def assemble(c_idx, buf):
            roff = pl.multiple_of(c_idx * CHUNK, CHUNK)

            def group(g, carry):
                s = pl.ds(pl.multiple_of(roff + g * 16, 16), 16)
                dt = b_v[s] - a_v[s]
                q = (dt * (N_INTERVAL / PASS_TIME)).astype(jnp.int32)
                idx = jnp.minimum(jnp.maximum(q, 0), N_INTERVAL)
                src = idx * OUT_DIM
                dst = row64 + g * (16 * OUT_DIM)
                nbuf = CHUNK * OUT_DIM
                srcs = [src + i for i in range(8)]
                dsts = [dst + i for i in range(8)]
                for c0 in range(0, OUT_DIM, 8):
                    tslice = tab_v.at[pl.ds(c0, tabn - 56)]
                    bslice = buf.at[pl.ds(c0, nbuf - 56)]
                    vals = [
                        plsc.load_gather(tslice, [srcs[i]]) for i in range(8)
                    ]
                    for i in range(8):
                        plsc.store_scatter(bslice, [dsts[i]], vals[i])
                return carry

            lax.fori_loop(0, NGROUP, group, 0)

        def start_store(c_idx, buf, sem):
            off = pl.multiple_of((base + c_idx * CHUNK) * OUT_DIM, CHUNK * OUT_DIM)
            return pltpu.async_copy(
                buf, out_h.at[pl.ds(off, CHUNK * OUT_DIM)], sem
            )

        def wait_store(buf, sem):
            pltpu.make_async_copy(
                buf, out_h.at[pl.ds(base * OUT_DIM, CHUNK * OUT_DIM)], sem
            ).wait()

        assemble(0, bufa)
        start_store(0, bufa, sema)
        assemble(1, bufb)
        start_store(1, bufb, semb)

        def pair(p, carry):
            wait_store(bufa, sema)
            assemble(2 * p, bufa)
            start_store(2 * p, bufa, sema)
            wait_store(bufb, semb)
            assemble(2 * p + 1, bufb)
            start_store(2 * p + 1, bufb, semb)
            return carry

        lax.fori_loop(1, npairs, pair, 0)
        wait_store(bufa, sema)
        wait_store(bufb, semb)

    return k(ts_a, ts_b, table_flat)


def kernel(inputs, timestamp, train, W, b):
    batch, L = inputs.shape
    rows = batch * L
    table = (W.T + b[None, :]).reshape(NBINS * OUT_DIM)
    ts_prev = timestamp[:, :-1]
    ts_a = ts_prev.reshape(rows)
    ts_b = timestamp[:, 1:].reshape(rows)
    out = _sc_time_encode(ts_a, ts_b, table, rows)
    return out.reshape(batch, L, OUT_DIM), ts_prev


# speedup vs baseline: 19.0661x; 2.6834x over previous
import functools

import jax
import jax.numpy as jnp
from jax import lax
from jax.experimental import pallas as pl
from jax.experimental.pallas import tpu as pltpu
from jax.experimental.pallas import tpu_sc as plsc

PASS_TIME = 1.0
N_INTERVAL = 100
OUT_DIM = 64
NBINS = N_INTERVAL + 1

NW = 32
CHUNK = 512
NGROUP = CHUNK // 16


def _sc_time_encode(ts_a, ts_b, table_flat, rows):
    rpw = rows // NW
    nchunks = rpw // CHUNK
    npairs = nchunks // 2
    tabn = NBINS * OUT_DIM
    mesh = plsc.VectorSubcoreMesh(core_axis_name="c", subcore_axis_name="s")

    @functools.partial(
        pl.kernel,
        mesh=mesh,
        out_type=jax.ShapeDtypeStruct((rows * OUT_DIM,), jnp.float32),
        scratch_types=[
            pltpu.VMEM((rpw,), jnp.float32),
            pltpu.VMEM((rpw,), jnp.float32),
            pltpu.VMEM((tabn,), jnp.float32),
            pltpu.VMEM((CHUNK * OUT_DIM,), jnp.float32),
            pltpu.VMEM((CHUNK * OUT_DIM,), jnp.float32),
            pltpu.SemaphoreType.DMA,
            pltpu.SemaphoreType.DMA,
        ],
        compiler_params=pltpu.CompilerParams(needs_layout_passes=False),
    )
    def k(a_h, b_h, tab_h, out_h, a_v, b_v, tab_v, bufa, bufb, sema, semb):
        wid = lax.axis_index("s") * 2 + lax.axis_index("c")
        base = pl.multiple_of(wid * rpw, rpw)
        pltpu.sync_copy(a_h.at[pl.ds(base, rpw)], a_v)
        pltpu.sync_copy(b_h.at[pl.ds(base, rpw)], b_v)
        pltpu.sync_copy(tab_h, tab_v)
        iota = lax.iota(jnp.int32, 16)
        row64 = iota * OUT_DIM
        offlow = [(iota + i) & 15 for i in range(16)]

        def assemble(c_idx, buf):
            roff = pl.multiple_of(c_idx * CHUNK, CHUNK)

            def group(g, carry):
                s = pl.ds(pl.multiple_of(roff + g * 16, 16), 16)
                dt = b_v[s] - a_v[s]
                q = (dt * (N_INTERVAL / PASS_TIME)).astype(jnp.int32)
                idx = jnp.minimum(jnp.maximum(q, 0), N_INTERVAL)
                src = idx * OUT_DIM
                dst = row64 + g * (16 * OUT_DIM)
                nbuf = CHUNK * OUT_DIM
                for c0 in range(0, OUT_DIM, 16):
                    tslice = tab_v.at[pl.ds(c0, tabn - 48)]
                    bslice = buf.at[pl.ds(c0, nbuf - 48)]
                    for i0 in range(0, 16, 8):
                        vals = [
                            plsc.load_gather(tslice, [src + offlow[i0 + i]])
                            for i in range(8)
                        ]
                        for i in range(8):
                            plsc.store_scatter(
                                bslice, [dst + offlow[i0 + i]], vals[i]
                            )
                return carry

            lax.fori_loop(0, NGROUP, group, 0)

        def start_store(c_idx, buf, sem):
            off = pl.multiple_of((base + c_idx * CHUNK) * OUT_DIM, CHUNK * OUT_DIM)
            return pltpu.async_copy(
                buf, out_h.at[pl.ds(off, CHUNK * OUT_DIM)], sem
            )

        def wait_store(buf, sem):
            pltpu.make_async_copy(
                buf, out_h.at[pl.ds(base * OUT_DIM, CHUNK * OUT_DIM)], sem
            ).wait()

        assemble(0, bufa)
        start_store(0, bufa, sema)
        assemble(1, bufb)
        start_store(1, bufb, semb)

        def pair(p, carry):
            wait_store(bufa, sema)
            assemble(2 * p, bufa)
            start_store(2 * p, bufa, sema)
            wait_store(bufb, semb)
            assemble(2 * p + 1, bufb)
            start_store(2 * p + 1, bufb, semb)
            return carry

        lax.fori_loop(1, npairs, pair, 0)
        wait_store(bufa, sema)
        wait_store(bufb, semb)

    return k(ts_a, ts_b, table_flat)


def kernel(inputs, timestamp, train, W, b):
    batch, L = inputs.shape
    rows = batch * L
    table = (W.T + b[None, :]).reshape(NBINS * OUT_DIM)
    ts_prev = timestamp[:, :-1]
    ts_a = ts_prev.reshape(rows)
    ts_b = timestamp[:, 1:].reshape(rows)
    out = _sc_time_encode(ts_a, ts_b, table, rows)
    return out.reshape(batch, L, OUT_DIM), ts_prev
